# baseline (speedup 1.0000x reference)
import functools

import jax
import jax.numpy as jnp
from jax import lax
from jax.experimental import pallas as pl

_ANCHORS = [1.3221, 1.73145, 3.19275, 4.00944, 5.05587, 8.09892, 9.47112,
            4.84053, 11.2364, 10.0071]
_AW = _ANCHORS[0::2]
_AH = _ANCHORS[1::2]
_NA = 5
_NC = 20
_NH = 19
_NW = 19
_NP = _NH * _NW
_NT = 50
_NF = 7 + _NC
_OBJ_SCALE = 5.0
_SIL = 0.6


def _loss_body(out_ref, tgt_ref, acc_ref):
    b = pl.program_id(0)
    blk = out_ref[0]
    tg = tgt_ref[0]

    f32 = jnp.float32
    t0 = tg[:, 0:1]
    t1 = tg[:, 1:2]
    gx = t1 * float(_NW)
    gy = tg[:, 2:3] * float(_NH)
    gw = tg[:, 3:4] * float(_NW)
    gh = tg[:, 4:5] * float(_NH)

    z = jnp.where(t1 == 0.0, 1.0, 0.0)
    zT = jnp.transpose(z)
    r_i = lax.broadcasted_iota(jnp.int32, (_NT, _NT), 0)
    c_i = lax.broadcasted_iota(jnp.int32, (_NT, _NT), 1)
    badcnt = jnp.sum(jnp.where(c_i <= r_i, zT + jnp.zeros((_NT, _NT), f32), 0.0),
                     axis=1, keepdims=True)
    valid = badcnt == 0.0
    validf = jnp.where(valid, 1.0, 0.0)

    def _anchor_iou(a):
        cwa = jnp.minimum(_AW[a], gw)
        cha = jnp.minimum(_AH[a], gh)
        inter_a = cwa * cha
        return jnp.where((cwa <= 0.0) | (cha <= 0.0), 0.0,
                         inter_a / (_AW[a] * _AH[a] + gw * gh - inter_a))

    best_val = _anchor_iou(0)
    best_idx = jnp.zeros((_NT, 1), jnp.int32)
    for a in range(1, _NA):
        cand = _anchor_iou(a)
        m = cand > best_val
        best_idx = jnp.where(m, a, best_idx)
        best_val = jnp.maximum(best_val, cand)
    n_w = jnp.where(best_val > 0.0, best_idx, _NA - 1)

    gi = gx.astype(jnp.int32)
    gj = gy.astype(jnp.int32)
    pidx = gj * _NW + gi
    tx_val = gx - gi.astype(f32)
    ty_val = gy - gj.astype(f32)

    aw_sel = jnp.full((_NT, 1), _AW[0], f32)
    ah_sel = jnp.full((_NT, 1), _AH[0], f32)
    for a in range(1, _NA):
        aw_sel = jnp.where(n_w == a, _AW[a], aw_sel)
        ah_sel = jnp.where(n_w == a, _AH[a], ah_sel)
    tw_val = jnp.log(gw / aw_sel)
    th_val = jnp.log(gh / ah_sel)
    clsidx = t0.astype(jnp.int32)

    cellid = (n_w * _NP + pidx).astype(f32)
    cellT = jnp.transpose(cellid)
    validT = jnp.transpose(validf)
    conflict = jnp.where((c_i > r_i) & (cellT == cellid) & (validT > 0.0),
                         1.0, 0.0)
    winner = valid & (jnp.sum(conflict, axis=1, keepdims=True) == 0.0)

    lane_p = lax.broadcasted_iota(jnp.int32, (1, _NP), 1)
    gridx = (lane_p % _NW).astype(f32)
    gridy = (lane_p // _NW).astype(f32)
    maskP = jnp.where(lane_p == pidx, 1.0, 0.0)

    gr = gx + gw * 0.5
    gl = gx - gw * 0.5
    gtp = gy + gh * 0.5
    gbt = gy - gh * 0.5
    gwz = gw * validf
    ghz = gh * validf
    gwgh = gw * gh

    dense = jnp.zeros((), f32)
    interc = jnp.zeros((_NT, 1), f32)
    rows = []
    for a in range(_NA):
        base = a * (5 + _NC)
        xr = blk[base + 0:base + 1, :]
        yr = blk[base + 1:base + 2, :]
        wr = blk[base + 2:base + 3, :]
        hr = blk[base + 3:base + 4, :]
        cr = blk[base + 4:base + 5, :]
        sx = jax.nn.sigmoid(xr)
        sy = jax.nn.sigmoid(yr)
        sc = jax.nn.sigmoid(cr)
        dense = dense + 0.5 * jnp.sum((sx - 0.5) ** 2 + (sy - 0.5) ** 2
                                      + wr * wr + hr * hr)

        px = sx + gridx
        py = sy + gridy
        pw = jnp.exp(wr) * _AW[a]
        ph = jnp.exp(hr) * _AH[a]
        pr = px + pw * 0.5
        pl_ = px - pw * 0.5
        pt = py + ph * 0.5
        pb = py - ph * 0.5
        pwph = pw * ph
        uw = jnp.maximum(pr, gr) - jnp.minimum(pl_, gl)
        uh = jnp.maximum(pt, gtp) - jnp.minimum(pb, gbt)
        cw = (pw + gwz) - uw
        chh = (ph + ghz) - uh
        inter = jnp.maximum(cw, 0.0) * jnp.maximum(chh, 0.0)
        denom = (pwph + gwgh) - inter
        excm = inter - _SIL * denom
        e01 = jnp.where(jnp.max(excm, axis=0, keepdims=True) > 0.0, 1.0, 0.0)
        dense = dense + 0.5 * jnp.sum((1.0 - e01) * sc * sc)
        interc = interc + jnp.where(
            n_w == a,
            jnp.sum(maskP * inter, axis=1, keepdims=True), 0.0)
        rows.append(jnp.concatenate(
            [sx, sy, wr, hr, sc, pwph, e01,
             blk[base + 5:base + 5 + _NC, :]], axis=0))

    bigF = jnp.concatenate(rows, axis=0)
    gall = lax.dot_general(maskP, bigF, (((1,), (1,)), ((), ())),
                           preferred_element_type=f32)
    gsel = gall[:, 0:_NF]
    for a in range(1, _NA):
        gsel = jnp.where(n_w == a, gall[:, a * _NF:(a + 1) * _NF], gsel)
    g_sx = gsel[:, 0:1]
    g_sy = gsel[:, 1:2]
    g_w = gsel[:, 2:3]
    g_h = gsel[:, 3:4]
    g_sc = gsel[:, 4:5]
    g_pwph = gsel[:, 5:6]
    cmb_cell = 1.0 - gsel[:, 6:7]
    glog = gsel[:, 7:_NF]

    tconf_val = interc / ((g_pwph + gwgh) - interc)
    gmax = jnp.max(glog, axis=1, keepdims=True)
    lse = jnp.log(jnp.sum(jnp.exp(glog - gmax), axis=1, keepdims=True)) + gmax
    lane_c = lax.broadcasted_iota(jnp.int32, (_NT, _NC), 1)
    picked = jnp.sum(jnp.where(lane_c == clsidx, glog, 0.0),
                     axis=1, keepdims=True)
    dcls = lse - picked

    delta = (0.5 * ((g_sx - tx_val) ** 2 - (g_sx - 0.5) ** 2)
             + 0.5 * ((g_sy - ty_val) ** 2 - (g_sy - 0.5) ** 2)
             + 0.5 * ((g_w - tw_val) ** 2 - g_w * g_w)
             + 0.5 * ((g_h - th_val) ** 2 - g_h * g_h)
             + 0.5 * (_OBJ_SCALE * (g_sc - tconf_val) ** 2
                      - cmb_cell * g_sc * g_sc)
             + dcls)
    sparse = jnp.sum(jnp.where(winner, delta, 0.0))

    @pl.when(b == 0)
    def _():
        acc_ref[:, :] = jnp.zeros((1, 1), f32)

    acc_ref[:, :] += jnp.reshape(dense + sparse, (1, 1))


@jax.jit
def kernel(output, target):
    nB = output.shape[0]
    outp = output.reshape(nB, _NA * (5 + _NC), _NP)
    tgt = target.reshape(nB, _NT, 5)
    res = pl.pallas_call(
        _loss_body,
        grid=(nB,),
        in_specs=[
            pl.BlockSpec((1, _NA * (5 + _NC), _NP), lambda b: (b, 0, 0)),
            pl.BlockSpec((1, _NT, 5), lambda b: (b, 0, 0)),
        ],
        out_specs=pl.BlockSpec((1, 1), lambda b: (0, 0)),
        out_shape=jax.ShapeDtypeStruct((1, 1), jnp.float32),
    )(outp, tgt)
    return res[0, 0]


# speedup vs baseline: 1.5683x; 1.5683x over previous
import functools

import jax
import jax.numpy as jnp
from jax import lax
from jax.experimental import pallas as pl

_ANCHORS = [1.3221, 1.73145, 3.19275, 4.00944, 5.05587, 8.09892, 9.47112,
            4.84053, 11.2364, 10.0071]
_AW = _ANCHORS[0::2]
_AH = _ANCHORS[1::2]
_NA = 5
_NC = 20
_NH = 19
_NW = 19
_NP = _NH * _NW
_NT = 50
_NF = 7 + _NC
_OBJ_SCALE = 5.0
_SIL = 0.6
_CB = 8


def _loss_body(out_ref, tgt_ref, tgtl_ref, acc_ref):
    g = pl.program_id(0)
    blk = out_ref[...]
    tg = tgt_ref[...]
    tgl = tgtl_ref[...]

    f32 = jnp.float32
    t0 = tg[:, :, 0:1]
    t1 = tg[:, :, 1:2]
    gx = t1 * float(_NW)
    gy = tg[:, :, 2:3] * float(_NH)
    gw = tg[:, :, 3:4] * float(_NW)
    gh = tg[:, :, 4:5] * float(_NH)
    t1L = tgl[:, 1:2, :]
    gwL = tgl[:, 3:4, :] * float(_NW)
    ghL = tgl[:, 4:5, :] * float(_NH)

    zS = jnp.where(t1 == 0.0, 1.0, 0.0)
    zL = jnp.where(t1L == 0.0, 1.0, 0.0)
    r_i = lax.broadcasted_iota(jnp.int32, (_CB, _NT, _NT), 1)
    c_i = lax.broadcasted_iota(jnp.int32, (_CB, _NT, _NT), 2)
    badS = jnp.sum(jnp.where(c_i <= r_i, zL + jnp.zeros((_CB, _NT, _NT), f32),
                             0.0), axis=2, keepdims=True)
    badL = jnp.sum(jnp.where(r_i <= c_i, zS + jnp.zeros((_CB, _NT, _NT), f32),
                             0.0), axis=1, keepdims=True)
    valid = badS == 0.0
    validf = jnp.where(valid, 1.0, 0.0)
    validL = jnp.where(badL == 0.0, 1.0, 0.0)

    def _best_anchor(w_, h_):
        best_val = None
        best_idx = jnp.zeros(w_.shape, jnp.int32)
        for a in range(_NA):
            cwa = jnp.minimum(_AW[a], w_)
            cha = jnp.minimum(_AH[a], h_)
            inter_a = cwa * cha
            iou = jnp.where((cwa <= 0.0) | (cha <= 0.0), 0.0,
                            inter_a / (_AW[a] * _AH[a] + w_ * h_ - inter_a))
            if best_val is None:
                best_val = iou
            else:
                m = iou > best_val
                best_idx = jnp.where(m, a, best_idx)
                best_val = jnp.maximum(best_val, iou)
        return jnp.where(best_val > 0.0, best_idx, _NA - 1)

    n_w = _best_anchor(gw, gh)
    n_wL = _best_anchor(gwL, ghL)

    gi = gx.astype(jnp.int32)
    gj = gy.astype(jnp.int32)
    pidx = gj * _NW + gi
    tx_val = gx - gi.astype(f32)
    ty_val = gy - gj.astype(f32)
    gxL = t1L * float(_NW)
    gyL = tgl[:, 2:3, :] * float(_NH)
    pidxL = gyL.astype(jnp.int32) * _NW + gxL.astype(jnp.int32)

    aw_sel = jnp.full((_CB, _NT, 1), _AW[0], f32)
    ah_sel = jnp.full((_CB, _NT, 1), _AH[0], f32)
    for a in range(1, _NA):
        aw_sel = jnp.where(n_w == a, _AW[a], aw_sel)
        ah_sel = jnp.where(n_w == a, _AH[a], ah_sel)
    tw_val = jnp.log(gw / aw_sel)
    th_val = jnp.log(gh / ah_sel)
    clsidx = t0.astype(jnp.int32)

    cellS = n_w * _NP + pidx
    cellL = n_wL * _NP + pidxL
    conflict = jnp.where((c_i > r_i) & (cellL == cellS) & (validL > 0.0),
                         1.0, 0.0)
    winner = valid & (jnp.sum(conflict, axis=2, keepdims=True) == 0.0)

    lane_p = lax.broadcasted_iota(jnp.int32, (_CB, 1, _NP), 2)
    gridx = (lane_p % _NW).astype(f32)
    gridy = (lane_p // _NW).astype(f32)
    maskP = jnp.where(lane_p == pidx, 1.0, 0.0)

    gr = gx + gw * 0.5
    gl = gx - gw * 0.5
    gtp = gy + gh * 0.5
    gbt = gy - gh * 0.5
    gwz = gw * validf
    ghz = gh * validf
    gwgh = gw * gh

    dense_acc = jnp.zeros((_CB, 1, _NP), f32)
    interc = jnp.zeros((_CB, _NT, 1), f32)
    rows = []
    for a in range(_NA):
        base = a * (5 + _NC)
        xr = blk[:, base + 0:base + 1, :]
        yr = blk[:, base + 1:base + 2, :]
        wr = blk[:, base + 2:base + 3, :]
        hr = blk[:, base + 3:base + 4, :]
        cr = blk[:, base + 4:base + 5, :]
        sx = jax.nn.sigmoid(xr)
        sy = jax.nn.sigmoid(yr)
        sc = jax.nn.sigmoid(cr)
        dense_acc = dense_acc + ((sx - 0.5) ** 2 + (sy - 0.5) ** 2
                                 + wr * wr + hr * hr)

        px = sx + gridx
        py = sy + gridy
        pw = jnp.exp(wr) * _AW[a]
        ph = jnp.exp(hr) * _AH[a]
        pr = px + pw * 0.5
        pl_ = px - pw * 0.5
        pt = py + ph * 0.5
        pb = py - ph * 0.5
        pwph = pw * ph
        uw = jnp.maximum(pr, gr) - jnp.minimum(pl_, gl)
        uh = jnp.maximum(pt, gtp) - jnp.minimum(pb, gbt)
        cw = (pw + gwz) - uw
        chh = (ph + ghz) - uh
        inter = jnp.maximum(cw, 0.0) * jnp.maximum(chh, 0.0)
        denom = (pwph + gwgh) - inter
        excm = inter - _SIL * denom
        e01 = jnp.where(jnp.max(excm, axis=1, keepdims=True) > 0.0, 1.0, 0.0)
        dense_acc = dense_acc + (1.0 - e01) * sc * sc
        interc = interc + jnp.where(
            n_w == a,
            jnp.sum(maskP * inter, axis=2, keepdims=True), 0.0)
        rows.append(jnp.concatenate(
            [sx, sy, wr, hr, sc, pwph, e01,
             blk[:, base + 5:base + 5 + _NC, :]], axis=1))

    dense = 0.5 * jnp.sum(dense_acc)

    bigF = jnp.concatenate(rows, axis=1)
    galls = []
    for q in range(_CB):
        galls.append(lax.dot_general(
            maskP[q], bigF[q], (((1,), (1,)), ((), ())),
            preferred_element_type=f32)[None])
    gall = jnp.concatenate(galls, axis=0)
    gsel = gall[:, :, 0:_NF]
    for a in range(1, _NA):
        gsel = jnp.where(n_w == a, gall[:, :, a * _NF:(a + 1) * _NF], gsel)
    g_sx = gsel[:, :, 0:1]
    g_sy = gsel[:, :, 1:2]
    g_w = gsel[:, :, 2:3]
    g_h = gsel[:, :, 3:4]
    g_sc = gsel[:, :, 4:5]
    g_pwph = gsel[:, :, 5:6]
    cmb_cell = 1.0 - gsel[:, :, 6:7]
    glog = gsel[:, :, 7:_NF]

    tconf_val = interc / ((g_pwph + gwgh) - interc)
    gmax = jnp.max(glog, axis=2, keepdims=True)
    lse = jnp.log(jnp.sum(jnp.exp(glog - gmax), axis=2, keepdims=True)) + gmax
    lane_c = lax.broadcasted_iota(jnp.int32, (_CB, _NT, _NC), 2)
    picked = jnp.sum(jnp.where(lane_c == clsidx, glog, 0.0),
                     axis=2, keepdims=True)
    dcls = lse - picked

    delta = (0.5 * ((g_sx - tx_val) ** 2 - (g_sx - 0.5) ** 2)
             + 0.5 * ((g_sy - ty_val) ** 2 - (g_sy - 0.5) ** 2)
             + 0.5 * ((g_w - tw_val) ** 2 - g_w * g_w)
             + 0.5 * ((g_h - th_val) ** 2 - g_h * g_h)
             + 0.5 * (_OBJ_SCALE * (g_sc - tconf_val) ** 2
                      - cmb_cell * g_sc * g_sc)
             + dcls)
    sparse = jnp.sum(jnp.where(winner, delta, 0.0))

    @pl.when(g == 0)
    def _():
        acc_ref[:, :] = jnp.zeros((1, 1), f32)

    acc_ref[:, :] += jnp.reshape(dense + sparse, (1, 1))


@jax.jit
def kernel(output, target):
    nB = output.shape[0]
    outp = output.reshape(nB, _NA * (5 + _NC), _NP)
    tgt = target.reshape(nB, _NT, 5)
    tgtl = jnp.swapaxes(tgt, 1, 2)
    res = pl.pallas_call(
        _loss_body,
        grid=(nB // _CB,),
        in_specs=[
            pl.BlockSpec((_CB, _NA * (5 + _NC), _NP), lambda g: (g, 0, 0)),
            pl.BlockSpec((_CB, _NT, 5), lambda g: (g, 0, 0)),
            pl.BlockSpec((_CB, 5, _NT), lambda g: (g, 0, 0)),
        ],
        out_specs=pl.BlockSpec((1, 1), lambda g: (0, 0)),
        out_shape=jax.ShapeDtypeStruct((1, 1), jnp.float32),
    )(outp, tgt, tgtl)
    return res[0, 0]


# speedup vs baseline: 1.6167x; 1.0309x over previous
import functools

import jax
import jax.numpy as jnp
from jax import lax
from jax.experimental import pallas as pl

_ANCHORS = [1.3221, 1.73145, 3.19275, 4.00944, 5.05587, 8.09892, 9.47112,
            4.84053, 11.2364, 10.0071]
_AW = _ANCHORS[0::2]
_AH = _ANCHORS[1::2]
_NA = 5
_NC = 20
_NH = 19
_NW = 19
_NP = _NH * _NW
_NT = 50
_OBJ_SCALE = 5.0
_SIL = 0.6
_CB = 8


def _loss_body(out_ref, tgt_ref, tgtl_ref, acc_ref):
    g = pl.program_id(0)
    blk = out_ref[...]
    tg = tgt_ref[...]
    tgl = tgtl_ref[...]

    f32 = jnp.float32
    bf16 = jnp.bfloat16
    t0 = tg[:, :, 0:1]
    t1 = tg[:, :, 1:2]
    gx = t1 * float(_NW)
    gy = tg[:, :, 2:3] * float(_NH)
    gw = tg[:, :, 3:4] * float(_NW)
    gh = tg[:, :, 4:5] * float(_NH)
    t1L = tgl[:, 1:2, :]
    gwL = tgl[:, 3:4, :] * float(_NW)
    ghL = tgl[:, 4:5, :] * float(_NH)

    zS = jnp.where(t1 == 0.0, 1.0, 0.0)
    zL = jnp.where(t1L == 0.0, 1.0, 0.0)
    r_i = lax.broadcasted_iota(jnp.int32, (_CB, _NT, _NT), 1)
    c_i = lax.broadcasted_iota(jnp.int32, (_CB, _NT, _NT), 2)
    badS = jnp.sum(jnp.where(c_i <= r_i, zL + jnp.zeros((_CB, _NT, _NT), f32),
                             0.0), axis=2, keepdims=True)
    badL = jnp.sum(jnp.where(r_i <= c_i, zS + jnp.zeros((_CB, _NT, _NT), f32),
                             0.0), axis=1, keepdims=True)
    valid = badS == 0.0
    validf = jnp.where(valid, 1.0, 0.0)
    validL = jnp.where(badL == 0.0, 1.0, 0.0)

    def _best_anchor(w_, h_):
        best_val = None
        best_idx = jnp.zeros(w_.shape, jnp.int32)
        for a in range(_NA):
            cwa = jnp.minimum(_AW[a], w_)
            cha = jnp.minimum(_AH[a], h_)
            inter_a = cwa * cha
            iou = jnp.where((cwa <= 0.0) | (cha <= 0.0), 0.0,
                            inter_a / (_AW[a] * _AH[a] + w_ * h_ - inter_a))
            if best_val is None:
                best_val = iou
            else:
                m = iou > best_val
                best_idx = jnp.where(m, a, best_idx)
                best_val = jnp.maximum(best_val, iou)
        return jnp.where(best_val > 0.0, best_idx, _NA - 1)

    n_w = _best_anchor(gw, gh)
    n_wL = _best_anchor(gwL, ghL)

    gi = gx.astype(jnp.int32)
    gj = gy.astype(jnp.int32)
    pidx = gj * _NW + gi
    gif = gi.astype(f32)
    gjf = gj.astype(f32)
    tx_val = gx - gif
    ty_val = gy - gjf
    gxL = t1L * float(_NW)
    gyL = tgl[:, 2:3, :] * float(_NH)
    pidxL = gyL.astype(jnp.int32) * _NW + gxL.astype(jnp.int32)

    aw_sel = jnp.full((_CB, _NT, 1), _AW[0], f32)
    ah_sel = jnp.full((_CB, _NT, 1), _AH[0], f32)
    for a in range(1, _NA):
        aw_sel = jnp.where(n_w == a, _AW[a], aw_sel)
        ah_sel = jnp.where(n_w == a, _AH[a], ah_sel)
    tw_val = jnp.log(gw / aw_sel)
    th_val = jnp.log(gh / ah_sel)
    clsidx = t0.astype(jnp.int32)

    cellS = n_w * _NP + pidx
    cellL = n_wL * _NP + pidxL
    conflict = jnp.where((c_i > r_i) & (cellL == cellS) & (validL > 0.0),
                         1.0, 0.0)
    winner = valid & (jnp.sum(conflict, axis=2, keepdims=True) == 0.0)

    lane_p = lax.broadcasted_iota(jnp.int32, (_CB, 1, _NP), 2)
    gridx = (lane_p % _NW).astype(f32)
    gridy = (lane_p // _NW).astype(f32)
    maskP = jnp.where(lane_p == pidx, 1.0, 0.0)

    gwz = gw * validf
    ghz = gh * validf
    grb = (gx + gw * 0.5).astype(bf16)
    glb = (gx - gw * 0.5).astype(bf16)
    gtb = (gy + gh * 0.5).astype(bf16)
    gbb = (gy - gh * 0.5).astype(bf16)
    gwb = gwz.astype(bf16)
    ghb = ghz.astype(bf16)
    gab = (gw * gh).astype(bf16)

    dense_acc = jnp.zeros((_CB, 1, _NP), f32)
    e_rows = []
    for a in range(_NA):
        base = a * (5 + _NC)
        xr = blk[:, base + 0:base + 1, :]
        yr = blk[:, base + 1:base + 2, :]
        wr = blk[:, base + 2:base + 3, :]
        hr = blk[:, base + 3:base + 4, :]
        cr = blk[:, base + 4:base + 5, :]
        sx = jax.nn.sigmoid(xr)
        sy = jax.nn.sigmoid(yr)
        sc = jax.nn.sigmoid(cr)
        dense_acc = dense_acc + ((sx - 0.5) ** 2 + (sy - 0.5) ** 2
                                 + wr * wr + hr * hr)

        pw = jnp.exp(wr) * _AW[a]
        ph = jnp.exp(hr) * _AH[a]
        px = sx + gridx
        py = sy + gridy
        prb = (px + pw * 0.5).astype(bf16)
        plb = (px - pw * 0.5).astype(bf16)
        ptb = (py + ph * 0.5).astype(bf16)
        pbb = (py - ph * 0.5).astype(bf16)
        pwb = pw.astype(bf16)
        phb = ph.astype(bf16)
        pab = (pw * ph).astype(bf16)
        uw = jnp.maximum(prb, grb) - jnp.minimum(plb, glb)
        uh = jnp.maximum(ptb, gtb) - jnp.minimum(pbb, gbb)
        cw = (pwb + gwb) - uw
        chh = (phb + ghb) - uh
        zb = jnp.zeros((), bf16)
        inter = jnp.maximum(cw, zb) * jnp.maximum(chh, zb)
        denom = (pab + gab) - inter
        excm = inter - bf16(_SIL) * denom
        em = jnp.max(excm, axis=1, keepdims=True).astype(f32)
        e01 = jnp.where(em > 0.0, 1.0, 0.0)
        dense_acc = dense_acc + (1.0 - e01) * sc * sc
        e_rows.append(e01)

    dense = 0.5 * jnp.sum(dense_acc)

    e5 = jnp.concatenate(e_rows, axis=1)
    graws = []
    ges = []
    for q in range(_CB):
        graws.append(lax.dot_general(
            maskP[q], blk[q], (((1,), (1,)), ((), ())),
            preferred_element_type=f32)[None])
        ges.append(lax.dot_general(
            maskP[q], e5[q], (((1,), (1,)), ((), ())),
            preferred_element_type=f32)[None])
    graw = jnp.concatenate(graws, axis=0)
    ge = jnp.concatenate(ges, axis=0)

    gsel = graw[:, :, 0:5 + _NC]
    e_cell = ge[:, :, 0:1]
    for a in range(1, _NA):
        sel = n_w == a
        gsel = jnp.where(sel, graw[:, :, a * 25:(a + 1) * 25], gsel)
        e_cell = jnp.where(sel, ge[:, :, a:a + 1], e_cell)
    cmb_cell = 1.0 - e_cell
    g_sx = jax.nn.sigmoid(gsel[:, :, 0:1])
    g_sy = jax.nn.sigmoid(gsel[:, :, 1:2])
    g_w = gsel[:, :, 2:3]
    g_h = gsel[:, :, 3:4]
    g_sc = jax.nn.sigmoid(gsel[:, :, 4:5])
    glog = gsel[:, :, 5:5 + _NC]

    pxc = g_sx + gif
    pyc = g_sy + gjf
    pwc = jnp.exp(g_w) * aw_sel
    phc = jnp.exp(g_h) * ah_sel
    uwc = jnp.maximum(pxc + pwc * 0.5, gx + gw * 0.5) - \
        jnp.minimum(pxc - pwc * 0.5, gx - gw * 0.5)
    uhc = jnp.maximum(pyc + phc * 0.5, gy + gh * 0.5) - \
        jnp.minimum(pyc - phc * 0.5, gy - gh * 0.5)
    cwc = (pwc + gw) - uwc
    chc = (phc + gh) - uhc
    interc = jnp.maximum(cwc, 0.0) * jnp.maximum(chc, 0.0)
    tconf_val = interc / ((pwc * phc + gw * gh) - interc)

    gmax = jnp.max(glog, axis=2, keepdims=True)
    lse = jnp.log(jnp.sum(jnp.exp(glog - gmax), axis=2, keepdims=True)) + gmax
    lane_c = lax.broadcasted_iota(jnp.int32, (_CB, _NT, _NC), 2)
    picked = jnp.sum(jnp.where(lane_c == clsidx, glog, 0.0),
                     axis=2, keepdims=True)
    dcls = lse - picked

    delta = (0.5 * ((g_sx - tx_val) ** 2 - (g_sx - 0.5) ** 2)
             + 0.5 * ((g_sy - ty_val) ** 2 - (g_sy - 0.5) ** 2)
             + 0.5 * ((g_w - tw_val) ** 2 - g_w * g_w)
             + 0.5 * ((g_h - th_val) ** 2 - g_h * g_h)
             + 0.5 * (_OBJ_SCALE * (g_sc - tconf_val) ** 2
                      - cmb_cell * g_sc * g_sc)
             + dcls)
    sparse = jnp.sum(jnp.where(winner, delta, 0.0))

    @pl.when(g == 0)
    def _():
        acc_ref[:, :] = jnp.zeros((1, 1), f32)

    acc_ref[:, :] += jnp.reshape(dense + sparse, (1, 1))


@jax.jit
def kernel(output, target):
    nB = output.shape[0]
    outp = output.reshape(nB, _NA * (5 + _NC), _NP)
    tgt = target.reshape(nB, _NT, 5)
    tgtl = jnp.swapaxes(tgt, 1, 2)
    res = pl.pallas_call(
        _loss_body,
        grid=(nB // _CB,),
        in_specs=[
            pl.BlockSpec((_CB, _NA * (5 + _NC), _NP), lambda g: (g, 0, 0)),
            pl.BlockSpec((_CB, _NT, 5), lambda g: (g, 0, 0)),
            pl.BlockSpec((_CB, 5, _NT), lambda g: (g, 0, 0)),
        ],
        out_specs=pl.BlockSpec((1, 1), lambda g: (0, 0)),
        out_shape=jax.ShapeDtypeStruct((1, 1), jnp.float32),
    )(outp, tgt, tgtl)
    return res[0, 0]


# speedup vs baseline: 2.7647x; 1.7101x over previous
import functools

import jax
import jax.numpy as jnp
from jax import lax
from jax.experimental import pallas as pl

_ANCHORS = [1.3221, 1.73145, 3.19275, 4.00944, 5.05587, 8.09892, 9.47112,
            4.84053, 11.2364, 10.0071]
_AW = _ANCHORS[0::2]
_AH = _ANCHORS[1::2]
_NA = 5
_NC = 20
_NH = 19
_NW = 19
_NP = _NH * _NW
_NT = 50
_OBJ_SCALE = 5.0
_SIL = 0.6
_CB = 8


def _loss_body(out_ref, tgt_ref, tgtl_ref, acc_ref):
    g = pl.program_id(0)
    blk = out_ref[...]
    tg = tgt_ref[...]
    tgl = tgtl_ref[...]

    f32 = jnp.float32
    bf16 = jnp.bfloat16

    t0L = tgl[:, 0:1, :]
    t1L = tgl[:, 1:2, :]
    gxL = t1L * float(_NW)
    gyL = tgl[:, 2:3, :] * float(_NH)
    gwL = tgl[:, 3:4, :] * float(_NW)
    ghL = tgl[:, 4:5, :] * float(_NH)

    t1 = tg[:, :, 1:2]
    gx = t1 * float(_NW)
    gy = tg[:, :, 2:3] * float(_NH)
    gw = tg[:, :, 3:4] * float(_NW)
    gh = tg[:, :, 4:5] * float(_NH)

    zS = jnp.where(t1 == 0.0, 1.0, 0.0)
    zL = jnp.where(t1L == 0.0, 1.0, 0.0)
    r_i = lax.broadcasted_iota(jnp.int32, (_CB, _NT, _NT), 1)
    c_i = lax.broadcasted_iota(jnp.int32, (_CB, _NT, _NT), 2)
    badS = jnp.sum(jnp.where(c_i <= r_i, zL + jnp.zeros((_CB, _NT, _NT), f32),
                             0.0), axis=2, keepdims=True)
    badL = jnp.sum(jnp.where(r_i <= c_i, zS + jnp.zeros((_CB, _NT, _NT), f32),
                             0.0), axis=1, keepdims=True)
    validS = jnp.where(badS == 0.0, 1.0, 0.0)
    validLb = badL == 0.0
    validL = jnp.where(validLb, 1.0, 0.0)

    best_val = None
    best_idx = jnp.zeros((_CB, 1, _NT), jnp.int32)
    for a in range(_NA):
        cwa = jnp.minimum(_AW[a], gwL)
        cha = jnp.minimum(_AH[a], ghL)
        inter_a = cwa * cha
        iou = jnp.where((cwa <= 0.0) | (cha <= 0.0), 0.0,
                        inter_a / (_AW[a] * _AH[a] + gwL * ghL - inter_a))
        if best_val is None:
            best_val = iou
        else:
            m = iou > best_val
            best_idx = jnp.where(m, a, best_idx)
            best_val = jnp.maximum(best_val, iou)
    n_wL = jnp.where(best_val > 0.0, best_idx, _NA - 1)

    giL = gxL.astype(jnp.int32)
    gjL = gyL.astype(jnp.int32)
    gifL = giL.astype(f32)
    gjfL = gjL.astype(f32)
    pidxL = gjL * _NW + giL
    tx_valL = gxL - gifL
    ty_valL = gyL - gjfL
    aw_selL = jnp.full((_CB, 1, _NT), _AW[0], f32)
    ah_selL = jnp.full((_CB, 1, _NT), _AH[0], f32)
    for a in range(1, _NA):
        aw_selL = jnp.where(n_wL == a, _AW[a], aw_selL)
        ah_selL = jnp.where(n_wL == a, _AH[a], ah_selL)
    tw_valL = jnp.log(gwL / aw_selL)
    th_valL = jnp.log(ghL / ah_selL)
    clsidxL = t0L.astype(jnp.int32)

    pidxS = (gy.astype(jnp.int32) * _NW + gx.astype(jnp.int32))
    diag = r_i == c_i
    n_wS = jnp.sum(jnp.where(diag, n_wL.astype(f32)
                             + jnp.zeros((_CB, _NT, _NT), f32), 0.0),
                   axis=2, keepdims=True).astype(jnp.int32)

    cellS = n_wS * _NP + pidxS
    cellL = n_wL * _NP + pidxL
    conflictL = jnp.where((r_i > c_i) & (cellS == cellL) & (validS > 0.0),
                          1.0, 0.0)
    winnerL = validLb & (jnp.sum(conflictL, axis=1, keepdims=True) == 0.0)

    lane_p = lax.broadcasted_iota(jnp.int32, (_CB, 1, _NP), 2)
    gridx = (lane_p % _NW).astype(f32)
    gridy = (lane_p // _NW).astype(f32)
    maskP = jnp.where(lane_p == pidxS, 1.0, 0.0)

    gwz = gw * validS
    ghz = gh * validS
    grb = (gx + gw * 0.5).astype(bf16)
    glb = (gx - gw * 0.5).astype(bf16)
    gtb = (gy + gh * 0.5).astype(bf16)
    gbb = (gy - gh * 0.5).astype(bf16)
    gwb = gwz.astype(bf16)
    ghb = ghz.astype(bf16)
    gab = (gw * gh).astype(bf16)

    dense_acc = jnp.zeros((_CB, 1, _NP), f32)
    e_rows = []
    for a in range(_NA):
        base = a * (5 + _NC)
        xr = blk[:, base + 0:base + 1, :]
        yr = blk[:, base + 1:base + 2, :]
        wr = blk[:, base + 2:base + 3, :]
        hr = blk[:, base + 3:base + 4, :]
        cr = blk[:, base + 4:base + 5, :]
        sx = jax.nn.sigmoid(xr)
        sy = jax.nn.sigmoid(yr)
        sc = jax.nn.sigmoid(cr)
        dense_acc = dense_acc + ((sx - 0.5) ** 2 + (sy - 0.5) ** 2
                                 + wr * wr + hr * hr)

        pw = jnp.exp(wr) * _AW[a]
        ph = jnp.exp(hr) * _AH[a]
        px = sx + gridx
        py = sy + gridy
        prb = (px + pw * 0.5).astype(bf16)
        plb = (px - pw * 0.5).astype(bf16)
        ptb = (py + ph * 0.5).astype(bf16)
        pbb = (py - ph * 0.5).astype(bf16)
        pwb = pw.astype(bf16)
        phb = ph.astype(bf16)
        pab = (pw * ph).astype(bf16)
        uw = jnp.maximum(prb, grb) - jnp.minimum(plb, glb)
        uh = jnp.maximum(ptb, gtb) - jnp.minimum(pbb, gbb)
        cw = (pwb + gwb) - uw
        chh = (phb + ghb) - uh
        zb = jnp.zeros((), bf16)
        inter = jnp.maximum(cw, zb) * jnp.maximum(chh, zb)
        denom = (pab + gab) - inter
        excm = inter - bf16(_SIL) * denom
        em = jnp.max(excm, axis=1, keepdims=True).astype(f32)
        e01 = jnp.where(em > 0.0, 1.0, 0.0)
        dense_acc = dense_acc + (1.0 - e01) * sc * sc
        e_rows.append(e01)

    dense = 0.5 * jnp.sum(dense_acc)

    e5 = jnp.concatenate(e_rows, axis=1)
    graws = []
    ges = []
    for q in range(_CB):
        graws.append(lax.dot_general(
            blk[q], maskP[q], (((1,), (1,)), ((), ())),
            preferred_element_type=f32)[None])
        ges.append(lax.dot_general(
            e5[q], maskP[q], (((1,), (1,)), ((), ())),
            preferred_element_type=f32)[None])
    graw = jnp.concatenate(graws, axis=0)
    ge = jnp.concatenate(ges, axis=0)

    gsel = graw[:, 0:5 + _NC, :]
    for a in range(1, _NA):
        gsel = jnp.where(n_wL == a, graw[:, a * 25:(a + 1) * 25, :], gsel)
    sub5 = lax.broadcasted_iota(jnp.int32, (_CB, _NA, _NT), 1)
    e_cell = jnp.sum(jnp.where(sub5 == n_wL, ge, 0.0),
                     axis=1, keepdims=True)
    cmb_cell = 1.0 - e_cell
    g_sx = jax.nn.sigmoid(gsel[:, 0:1, :])
    g_sy = jax.nn.sigmoid(gsel[:, 1:2, :])
    g_w = gsel[:, 2:3, :]
    g_h = gsel[:, 3:4, :]
    g_sc = jax.nn.sigmoid(gsel[:, 4:5, :])
    glog = gsel[:, 5:5 + _NC, :]

    pxc = g_sx + gifL
    pyc = g_sy + gjfL
    pwc = jnp.exp(g_w) * aw_selL
    phc = jnp.exp(g_h) * ah_selL
    uwc = jnp.maximum(pxc + pwc * 0.5, gxL + gwL * 0.5) - \
        jnp.minimum(pxc - pwc * 0.5, gxL - gwL * 0.5)
    uhc = jnp.maximum(pyc + phc * 0.5, gyL + ghL * 0.5) - \
        jnp.minimum(pyc - phc * 0.5, gyL - ghL * 0.5)
    cwc = (pwc + gwL) - uwc
    chc = (phc + ghL) - uhc
    interc = jnp.maximum(cwc, 0.0) * jnp.maximum(chc, 0.0)
    tconf_val = interc / ((pwc * phc + gwL * ghL) - interc)

    gmax = jnp.max(glog, axis=1, keepdims=True)
    lse = jnp.log(jnp.sum(jnp.exp(glog - gmax), axis=1, keepdims=True)) + gmax
    sub_c = lax.broadcasted_iota(jnp.int32, (_CB, _NC, _NT), 1)
    picked = jnp.sum(jnp.where(sub_c == clsidxL, glog, 0.0),
                     axis=1, keepdims=True)
    dcls = lse - picked

    delta = (0.5 * ((g_sx - tx_valL) ** 2 - (g_sx - 0.5) ** 2)
             + 0.5 * ((g_sy - ty_valL) ** 2 - (g_sy - 0.5) ** 2)
             + 0.5 * ((g_w - tw_valL) ** 2 - g_w * g_w)
             + 0.5 * ((g_h - th_valL) ** 2 - g_h * g_h)
             + 0.5 * (_OBJ_SCALE * (g_sc - tconf_val) ** 2
                      - cmb_cell * g_sc * g_sc)
             + dcls)
    sparse = jnp.sum(jnp.where(winnerL, delta, 0.0))

    @pl.when(g == 0)
    def _():
        acc_ref[:, :] = jnp.zeros((1, 1), f32)

    acc_ref[:, :] += jnp.reshape(dense + sparse, (1, 1))


@jax.jit
def kernel(output, target):
    nB = output.shape[0]
    outp = output.reshape(nB, _NA * (5 + _NC), _NP)
    tgt = target.reshape(nB, _NT, 5)
    tgtl = jnp.swapaxes(tgt, 1, 2)
    res = pl.pallas_call(
        _loss_body,
        grid=(nB // _CB,),
        in_specs=[
            pl.BlockSpec((_CB, _NA * (5 + _NC), _NP), lambda g: (g, 0, 0)),
            pl.BlockSpec((_CB, _NT, 5), lambda g: (g, 0, 0)),
            pl.BlockSpec((_CB, 5, _NT), lambda g: (g, 0, 0)),
        ],
        out_specs=pl.BlockSpec((1, 1), lambda g: (0, 0)),
        out_shape=jax.ShapeDtypeStruct((1, 1), jnp.float32),
    )(outp, tgt, tgtl)
    return res[0, 0]


# speedup vs baseline: 2.7815x; 1.0061x over previous
import functools

import jax
import jax.numpy as jnp
from jax import lax
from jax.experimental import pallas as pl

_ANCHORS = [1.3221, 1.73145, 3.19275, 4.00944, 5.05587, 8.09892, 9.47112,
            4.84053, 11.2364, 10.0071]
_AW = _ANCHORS[0::2]
_AH = _ANCHORS[1::2]
_NA = 5
_NC = 20
_NH = 19
_NW = 19
_NP = _NH * _NW
_NT = 50
_OBJ_SCALE = 5.0
_SIL = 0.6
_CB = 16


def _loss_body(out_ref, tgt_ref, tgtl_ref, acc_ref):
    g = pl.program_id(0)
    blk = out_ref[...]
    tg = tgt_ref[...]
    tgl = tgtl_ref[...]

    f32 = jnp.float32
    bf16 = jnp.bfloat16

    t0L = tgl[:, 0:1, :]
    t1L = tgl[:, 1:2, :]
    gxL = t1L * float(_NW)
    gyL = tgl[:, 2:3, :] * float(_NH)
    gwL = tgl[:, 3:4, :] * float(_NW)
    ghL = tgl[:, 4:5, :] * float(_NH)

    t1 = tg[:, :, 1:2]
    gx = t1 * float(_NW)
    gy = tg[:, :, 2:3] * float(_NH)
    gw = tg[:, :, 3:4] * float(_NW)
    gh = tg[:, :, 4:5] * float(_NH)

    zS = jnp.where(t1 == 0.0, 1.0, 0.0)
    zL = jnp.where(t1L == 0.0, 1.0, 0.0)
    r_i = lax.broadcasted_iota(jnp.int32, (_CB, _NT, _NT), 1)
    c_i = lax.broadcasted_iota(jnp.int32, (_CB, _NT, _NT), 2)
    badS = jnp.sum(jnp.where(c_i <= r_i, zL + jnp.zeros((_CB, _NT, _NT), f32),
                             0.0), axis=2, keepdims=True)
    badL = jnp.sum(jnp.where(r_i <= c_i, zS + jnp.zeros((_CB, _NT, _NT), f32),
                             0.0), axis=1, keepdims=True)
    validS = jnp.where(badS == 0.0, 1.0, 0.0)
    validLb = badL == 0.0
    validL = jnp.where(validLb, 1.0, 0.0)

    best_val = None
    best_idx = jnp.zeros((_CB, 1, _NT), jnp.int32)
    for a in range(_NA):
        cwa = jnp.minimum(_AW[a], gwL)
        cha = jnp.minimum(_AH[a], ghL)
        inter_a = cwa * cha
        iou = jnp.where((cwa <= 0.0) | (cha <= 0.0), 0.0,
                        inter_a / (_AW[a] * _AH[a] + gwL * ghL - inter_a))
        if best_val is None:
            best_val = iou
        else:
            m = iou > best_val
            best_idx = jnp.where(m, a, best_idx)
            best_val = jnp.maximum(best_val, iou)
    n_wL = jnp.where(best_val > 0.0, best_idx, _NA - 1)

    giL = gxL.astype(jnp.int32)
    gjL = gyL.astype(jnp.int32)
    gifL = giL.astype(f32)
    gjfL = gjL.astype(f32)
    pidxL = gjL * _NW + giL
    tx_valL = gxL - gifL
    ty_valL = gyL - gjfL
    aw_selL = jnp.full((_CB, 1, _NT), _AW[0], f32)
    ah_selL = jnp.full((_CB, 1, _NT), _AH[0], f32)
    for a in range(1, _NA):
        aw_selL = jnp.where(n_wL == a, _AW[a], aw_selL)
        ah_selL = jnp.where(n_wL == a, _AH[a], ah_selL)
    tw_valL = jnp.log(gwL / aw_selL)
    th_valL = jnp.log(ghL / ah_selL)
    clsidxL = t0L.astype(jnp.int32)

    pidxS = (gy.astype(jnp.int32) * _NW + gx.astype(jnp.int32))
    diag = r_i == c_i
    n_wS = jnp.sum(jnp.where(diag, n_wL.astype(f32)
                             + jnp.zeros((_CB, _NT, _NT), f32), 0.0),
                   axis=2, keepdims=True).astype(jnp.int32)

    cellS = n_wS * _NP + pidxS
    cellL = n_wL * _NP + pidxL
    conflictL = jnp.where((r_i > c_i) & (cellS == cellL) & (validS > 0.0),
                          1.0, 0.0)
    winnerL = validLb & (jnp.sum(conflictL, axis=1, keepdims=True) == 0.0)

    lane_p = lax.broadcasted_iota(jnp.int32, (_CB, 1, _NP), 2)
    gridx = (lane_p % _NW).astype(f32)
    gridy = (lane_p // _NW).astype(f32)
    maskP = jnp.where(lane_p == pidxS, 1.0, 0.0)

    gwz = gw * validS
    ghz = gh * validS
    grb = (gx + gw * 0.5).astype(bf16)
    glb = (gx - gw * 0.5).astype(bf16)
    gtb = (gy + gh * 0.5).astype(bf16)
    gbb = (gy - gh * 0.5).astype(bf16)
    gwb = gwz.astype(bf16)
    ghb = ghz.astype(bf16)
    gab = (gw * gh).astype(bf16)

    dense_acc = jnp.zeros((_CB, 1, _NP), f32)
    e_rows = []
    for a in range(_NA):
        base = a * (5 + _NC)
        xr = blk[:, base + 0:base + 1, :]
        yr = blk[:, base + 1:base + 2, :]
        wr = blk[:, base + 2:base + 3, :]
        hr = blk[:, base + 3:base + 4, :]
        cr = blk[:, base + 4:base + 5, :]
        sx = jax.nn.sigmoid(xr)
        sy = jax.nn.sigmoid(yr)
        sc = jax.nn.sigmoid(cr)
        dense_acc = dense_acc + ((sx - 0.5) ** 2 + (sy - 0.5) ** 2
                                 + wr * wr + hr * hr)

        pw = jnp.exp(wr) * _AW[a]
        ph = jnp.exp(hr) * _AH[a]
        px = sx + gridx
        py = sy + gridy
        prb = (px + pw * 0.5).astype(bf16)
        plb = (px - pw * 0.5).astype(bf16)
        ptb = (py + ph * 0.5).astype(bf16)
        pbb = (py - ph * 0.5).astype(bf16)
        pwb = pw.astype(bf16)
        phb = ph.astype(bf16)
        pab = (pw * ph).astype(bf16)
        uw = jnp.maximum(prb, grb) - jnp.minimum(plb, glb)
        uh = jnp.maximum(ptb, gtb) - jnp.minimum(pbb, gbb)
        cw = (pwb + gwb) - uw
        chh = (phb + ghb) - uh
        zb = jnp.zeros((), bf16)
        inter = jnp.maximum(cw, zb) * jnp.maximum(chh, zb)
        denom = (pab + gab) - inter
        excm = inter - bf16(_SIL) * denom
        em = jnp.max(excm, axis=1, keepdims=True).astype(f32)
        e01 = jnp.where(em > 0.0, 1.0, 0.0)
        dense_acc = dense_acc + (1.0 - e01) * sc * sc
        e_rows.append(e01)

    dense = 0.5 * jnp.sum(dense_acc)

    e5 = jnp.concatenate(e_rows, axis=1)
    graws = []
    ges = []
    for q in range(_CB):
        graws.append(lax.dot_general(
            blk[q], maskP[q], (((1,), (1,)), ((), ())),
            preferred_element_type=f32)[None])
        ges.append(lax.dot_general(
            e5[q], maskP[q], (((1,), (1,)), ((), ())),
            preferred_element_type=f32)[None])
    graw = jnp.concatenate(graws, axis=0)
    ge = jnp.concatenate(ges, axis=0)

    gsel = graw[:, 0:5 + _NC, :]
    for a in range(1, _NA):
        gsel = jnp.where(n_wL == a, graw[:, a * 25:(a + 1) * 25, :], gsel)
    sub5 = lax.broadcasted_iota(jnp.int32, (_CB, _NA, _NT), 1)
    e_cell = jnp.sum(jnp.where(sub5 == n_wL, ge, 0.0),
                     axis=1, keepdims=True)
    cmb_cell = 1.0 - e_cell
    g_sx = jax.nn.sigmoid(gsel[:, 0:1, :])
    g_sy = jax.nn.sigmoid(gsel[:, 1:2, :])
    g_w = gsel[:, 2:3, :]
    g_h = gsel[:, 3:4, :]
    g_sc = jax.nn.sigmoid(gsel[:, 4:5, :])
    glog = gsel[:, 5:5 + _NC, :]

    pxc = g_sx + gifL
    pyc = g_sy + gjfL
    pwc = jnp.exp(g_w) * aw_selL
    phc = jnp.exp(g_h) * ah_selL
    uwc = jnp.maximum(pxc + pwc * 0.5, gxL + gwL * 0.5) - \
        jnp.minimum(pxc - pwc * 0.5, gxL - gwL * 0.5)
    uhc = jnp.maximum(pyc + phc * 0.5, gyL + ghL * 0.5) - \
        jnp.minimum(pyc - phc * 0.5, gyL - ghL * 0.5)
    cwc = (pwc + gwL) - uwc
    chc = (phc + ghL) - uhc
    interc = jnp.maximum(cwc, 0.0) * jnp.maximum(chc, 0.0)
    tconf_val = interc / ((pwc * phc + gwL * ghL) - interc)

    gmax = jnp.max(glog, axis=1, keepdims=True)
    lse = jnp.log(jnp.sum(jnp.exp(glog - gmax), axis=1, keepdims=True)) + gmax
    sub_c = lax.broadcasted_iota(jnp.int32, (_CB, _NC, _NT), 1)
    picked = jnp.sum(jnp.where(sub_c == clsidxL, glog, 0.0),
                     axis=1, keepdims=True)
    dcls = lse - picked

    delta = (0.5 * ((g_sx - tx_valL) ** 2 - (g_sx - 0.5) ** 2)
             + 0.5 * ((g_sy - ty_valL) ** 2 - (g_sy - 0.5) ** 2)
             + 0.5 * ((g_w - tw_valL) ** 2 - g_w * g_w)
             + 0.5 * ((g_h - th_valL) ** 2 - g_h * g_h)
             + 0.5 * (_OBJ_SCALE * (g_sc - tconf_val) ** 2
                      - cmb_cell * g_sc * g_sc)
             + dcls)
    sparse = jnp.sum(jnp.where(winnerL, delta, 0.0))

    @pl.when(g == 0)
    def _():
        acc_ref[:, :] = jnp.zeros((1, 1), f32)

    acc_ref[:, :] += jnp.reshape(dense + sparse, (1, 1))


@jax.jit
def kernel(output, target):
    nB = output.shape[0]
    outp = output.reshape(nB, _NA * (5 + _NC), _NP)
    tgt = target.reshape(nB, _NT, 5)
    tgtl = jnp.swapaxes(tgt, 1, 2)
    res = pl.pallas_call(
        _loss_body,
        grid=(nB // _CB,),
        in_specs=[
            pl.BlockSpec((_CB, _NA * (5 + _NC), _NP), lambda g: (g, 0, 0)),
            pl.BlockSpec((_CB, _NT, 5), lambda g: (g, 0, 0)),
            pl.BlockSpec((_CB, 5, _NT), lambda g: (g, 0, 0)),
        ],
        out_specs=pl.BlockSpec((1, 1), lambda g: (0, 0)),
        out_shape=jax.ShapeDtypeStruct((1, 1), jnp.float32),
    )(outp, tgt, tgtl)
    return res[0, 0]
